# baseline (device time: 197901 ns/iter reference)
import jax
import jax.numpy as jnp
from jax import lax
from jax.experimental import pallas as pl
from jax.experimental.pallas import tpu as pltpu

N_DEV = 16
SCALE = 0.08838834764831843
BLK = 64
QC = 512


def kernel(x, Wq, K_ext, V_ext, Wo):
    B, Sq, Dm = x.shape
    _, HLDh = Wq.shape
    _, Skv, Hq_g, Dh = K_ext.shape
    HL = HLDh // Dh
    NQC = Sq // QC

    def compute_body(x_ref, wq_ref, k_hbm, v_hbm, wo_ref, out_ref,
                     q_scr, k_scr, v_scr, sems):
        my_i = lax.axis_index("i")
        x2d = x_ref[0]
        q_scr[...] = jnp.dot(x2d, wq_ref[...],
                             preferred_element_type=jnp.float32)
        out_ref[...] = jnp.zeros_like(out_ref)
        for h in range(HL):
            head = my_i * HL + h
            ck = pltpu.make_async_copy(
                k_hbm.at[0, :, pl.ds(head, 1), :], k_scr, sems.at[0])
            cv = pltpu.make_async_copy(
                v_hbm.at[0, :, pl.ds(head, 1), :], v_scr, sems.at[1])
            ck.start()
            cv.start()
            ck.wait()
            cv.wait()
            kh = k_scr[:, 0, :]
            vh = v_scr[:, 0, :]
            wo_blk = wo_ref[h * Dh:(h + 1) * Dh, :]
            for qc in range(NQC):
                kmax = (qc + 1) * QC
                qh = q_scr[qc * QC:(qc + 1) * QC, h * Dh:(h + 1) * Dh]
                s = lax.dot_general(
                    qh, kh[:kmax, :], (((1,), (1,)), ((), ())),
                    preferred_element_type=jnp.float32) * SCALE
                rows = (lax.broadcasted_iota(jnp.int32, (QC, kmax), 0)
                        + qc * QC) // BLK
                cols = lax.broadcasted_iota(jnp.int32, (QC, kmax), 1) // BLK
                s = jnp.where(cols <= rows, s, -1e9)
                m = jnp.max(s, axis=-1, keepdims=True)
                w = jnp.exp(s - m)
                w = w / jnp.sum(w, axis=-1, keepdims=True)
                ctx = jnp.dot(w, vh[:kmax, :],
                              preferred_element_type=jnp.float32)
                out_ref[0, qc * QC:(qc + 1) * QC, :] = (
                    out_ref[0, qc * QC:(qc + 1) * QC, :]
                    + jnp.dot(ctx, wo_blk,
                              preferred_element_type=jnp.float32))

    partial = pl.pallas_call(
        compute_body,
        out_shape=jax.ShapeDtypeStruct((B, Sq, Dm), jnp.float32),
        in_specs=[
            pl.BlockSpec(memory_space=pltpu.VMEM),
            pl.BlockSpec(memory_space=pltpu.VMEM),
            pl.BlockSpec(memory_space=pltpu.MemorySpace.HBM),
            pl.BlockSpec(memory_space=pltpu.MemorySpace.HBM),
            pl.BlockSpec(memory_space=pltpu.VMEM),
        ],
        out_specs=pl.BlockSpec(memory_space=pltpu.VMEM),
        scratch_shapes=[
            pltpu.VMEM((Sq, HLDh), jnp.float32),
            pltpu.VMEM((Skv, 1, Dh), jnp.float32),
            pltpu.VMEM((Skv, 1, Dh), jnp.float32),
            pltpu.SemaphoreType.DMA((2,)),
        ],
    )(x, Wq, K_ext, V_ext, Wo)

    CH = Sq // N_DEV
    HD = Dm // 2

    def ar_body(p_ref, out_ref,
                rsbufR, sendbufR, agbufR, redbufR,
                rsbufL, sendbufL, agbufL, redbufL,
                pbufR, pbufL,
                rs_ssR, rs_rsR, ag_ssR, ag_rsR,
                rs_ssL, rs_rsL, ag_ssL, ag_rsL):
        my = lax.axis_index("i")
        z = my // 4
        p = my % 4
        pos = jnp.where(p == 0, z,
              jnp.where(p == 3, 7 - z,
              jnp.where(p == 2, 8 + z, 15 - z)))

        def ring_logical(s):
            s = s % N_DEV
            seg = s // 4
            k = s % 4
            return jnp.where(seg == 0, 4 * k,
                   jnp.where(seg == 1, 4 * (3 - k) + 3,
                   jnp.where(seg == 2, 4 * k + 2, 4 * (3 - k) + 1)))

        left = ring_logical(pos + N_DEV - 1)
        right = ring_logical(pos + 1)

        barrier_sem = pltpu.get_barrier_semaphore()
        for nbr in (left, right):
            pl.semaphore_signal(
                barrier_sem, inc=1,
                device_id=(nbr,), device_id_type=pl.DeviceIdType.MESH)
        pl.semaphore_wait(barrier_sem, 2)

        bf16 = jnp.bfloat16
        f32 = jnp.float32
        pbufR[...] = p_ref[0, :, 0:HD].astype(bf16)
        pbufL[...] = p_ref[0, :, HD:Dm].astype(bf16)

        def rs_rdma(h):
            r = pltpu.make_async_remote_copy(
                src_ref=sendbufR.at[h], dst_ref=rsbufR.at[h],
                send_sem=rs_ssR.at[h], recv_sem=rs_rsR.at[h],
                device_id=(right,), device_id_type=pl.DeviceIdType.MESH)
            l = pltpu.make_async_remote_copy(
                src_ref=sendbufL.at[h], dst_ref=rsbufL.at[h],
                send_sem=rs_ssL.at[h], recv_sem=rs_rsL.at[h],
                device_id=(left,), device_id_type=pl.DeviceIdType.MESH)
            return r, l

        rs_desc = []
        sendbufR[0, :, :] = pbufR[pl.ds(pos * CH, CH), :]
        sendbufL[0, :, :] = pbufL[pl.ds(pos * CH, CH), :]
        d = rs_rdma(0)
        d[0].start()
        d[1].start()
        rs_desc.append(d)
        for h in range(N_DEV - 1):
            rs_desc[h][0].wait_recv()
            rs_desc[h][1].wait_recv()
            ckR = (pos + 2 * N_DEV - 1 - h) % N_DEV
            ckL = (pos + 1 + h) % N_DEV
            if h < N_DEV - 2:
                sendbufR[h + 1, :, :] = (
                    rsbufR[h] + pbufR[pl.ds(ckR * CH, CH), :])
                sendbufL[h + 1, :, :] = (
                    rsbufL[h] + pbufL[pl.ds(ckL * CH, CH), :])
                d = rs_rdma(h + 1)
                d[0].start()
                d[1].start()
                rs_desc.append(d)
            else:
                accR = (rsbufR[h].astype(f32)
                        + p_ref[0, pl.ds(ckR * CH, CH), 0:HD])
                accL = (rsbufL[h].astype(f32)
                        + p_ref[0, pl.ds(ckL * CH, CH), HD:Dm])
                redbufR[...] = accR.astype(bf16)
                redbufL[...] = accL.astype(bf16)

        def ag_rdma(h):
            r = pltpu.make_async_remote_copy(
                src_ref=(redbufR if h == 0 else agbufR.at[h - 1]),
                dst_ref=agbufR.at[h],
                send_sem=ag_ssR.at[h], recv_sem=ag_rsR.at[h],
                device_id=(right,), device_id_type=pl.DeviceIdType.MESH)
            l = pltpu.make_async_remote_copy(
                src_ref=(redbufL if h == 0 else agbufL.at[h - 1]),
                dst_ref=agbufL.at[h],
                send_sem=ag_ssL.at[h], recv_sem=ag_rsL.at[h],
                device_id=(left,), device_id_type=pl.DeviceIdType.MESH)
            return r, l

        ag_desc = []
        d = ag_rdma(0)
        d[0].start()
        d[1].start()
        ag_desc.append(d)
        ownR = (pos + 1) % N_DEV
        ownL = (pos + N_DEV - 1) % N_DEV
        out_ref[0, pl.ds(ownR * CH, CH), 0:HD] = accR
        out_ref[0, pl.ds(ownL * CH, CH), HD:Dm] = accL
        for h in range(N_DEV - 1):
            ag_desc[h][0].wait_recv()
            ag_desc[h][1].wait_recv()
            if h < N_DEV - 2:
                d = ag_rdma(h + 1)
                d[0].start()
                d[1].start()
                ag_desc.append(d)
            idxR = (pos + 2 * N_DEV - h) % N_DEV
            idxL = (pos + h) % N_DEV
            out_ref[0, pl.ds(idxR * CH, CH), 0:HD] = agbufR[h].astype(f32)
            out_ref[0, pl.ds(idxL * CH, CH), HD:Dm] = agbufL[h].astype(f32)

        for dR, dL in rs_desc + ag_desc:
            dR.wait_send()
            dL.wait_send()

    nh = N_DEV - 1
    return pl.pallas_call(
        ar_body,
        out_shape=jax.ShapeDtypeStruct((B, Sq, Dm), jnp.float32),
        in_specs=[pl.BlockSpec(memory_space=pltpu.VMEM)],
        out_specs=pl.BlockSpec(memory_space=pltpu.VMEM),
        scratch_shapes=[
            pltpu.VMEM((nh, CH, HD), jnp.bfloat16),
            pltpu.VMEM((nh, CH, HD), jnp.bfloat16),
            pltpu.VMEM((nh, CH, HD), jnp.bfloat16),
            pltpu.VMEM((CH, HD), jnp.bfloat16),
            pltpu.VMEM((nh, CH, HD), jnp.bfloat16),
            pltpu.VMEM((nh, CH, HD), jnp.bfloat16),
            pltpu.VMEM((nh, CH, HD), jnp.bfloat16),
            pltpu.VMEM((CH, HD), jnp.bfloat16),
            pltpu.VMEM((Sq, HD), jnp.bfloat16),
            pltpu.VMEM((Sq, HD), jnp.bfloat16),
            pltpu.SemaphoreType.DMA((nh,)),
            pltpu.SemaphoreType.DMA((nh,)),
            pltpu.SemaphoreType.DMA((nh,)),
            pltpu.SemaphoreType.DMA((nh,)),
            pltpu.SemaphoreType.DMA((nh,)),
            pltpu.SemaphoreType.DMA((nh,)),
            pltpu.SemaphoreType.DMA((nh,)),
            pltpu.SemaphoreType.DMA((nh,)),
        ],
        compiler_params=pltpu.CompilerParams(collective_id=0),
    )(partial)


# device time: 197205 ns/iter; 1.0035x vs baseline; 1.0035x over previous
import jax
import jax.numpy as jnp
from jax import lax
from jax.experimental import pallas as pl
from jax.experimental.pallas import tpu as pltpu

N_DEV = 16
SCALE = 0.08838834764831843
BLK = 64
QC = 512


def kernel(x, Wq, K_ext, V_ext, Wo):
    B, Sq, Dm = x.shape
    _, HLDh = Wq.shape
    _, Skv, Hq_g, Dh = K_ext.shape
    HL = HLDh // Dh
    NQC = Sq // QC

    def compute_body(x_ref, wq_ref, k_hbm, v_hbm, wo_ref, out_ref,
                     q_scr, k_scr, v_scr, ksems, vsems):
        my_i = lax.axis_index("i")

        def start_kv(hh):
            slot = hh % 2
            head = my_i * HL + hh
            ck = pltpu.make_async_copy(
                k_hbm.at[0, :, pl.ds(head, 1), :], k_scr.at[slot],
                ksems.at[slot])
            cv = pltpu.make_async_copy(
                v_hbm.at[0, :, pl.ds(head, 1), :], v_scr.at[slot],
                vsems.at[slot])
            ck.start()
            cv.start()
            return ck, cv

        x2d = x_ref[0]
        q_scr[...] = jnp.dot(x2d, wq_ref[...],
                             preferred_element_type=jnp.float32)
        out_ref[...] = jnp.zeros_like(out_ref)
        for h in range(HL):
            ck, cv = start_kv(h)
            ck.wait()
            cv.wait()
            slot = h % 2
            kh = k_scr[slot, :, 0, :]
            vh = v_scr[slot, :, 0, :]
            wo_blk = wo_ref[h * Dh:(h + 1) * Dh, :]
            for qc in range(NQC):
                kmax = (qc + 1) * QC
                qh = q_scr[qc * QC:(qc + 1) * QC, h * Dh:(h + 1) * Dh]
                s = lax.dot_general(
                    qh, kh[:kmax, :], (((1,), (1,)), ((), ())),
                    preferred_element_type=jnp.float32) * SCALE
                rows = (lax.broadcasted_iota(jnp.int32, (QC, kmax), 0)
                        + qc * QC) // BLK
                cols = lax.broadcasted_iota(jnp.int32, (QC, kmax), 1) // BLK
                s = jnp.where(cols <= rows, s, -1e9)
                m = jnp.max(s, axis=-1, keepdims=True)
                w = jnp.exp(s - m)
                w = w / jnp.sum(w, axis=-1, keepdims=True)
                ctx = jnp.dot(w, vh[:kmax, :],
                              preferred_element_type=jnp.float32)
                out_ref[0, qc * QC:(qc + 1) * QC, :] = (
                    out_ref[0, qc * QC:(qc + 1) * QC, :]
                    + jnp.dot(ctx, wo_blk,
                              preferred_element_type=jnp.float32))

    partial = pl.pallas_call(
        compute_body,
        out_shape=jax.ShapeDtypeStruct((B, Sq, Dm), jnp.float32),
        in_specs=[
            pl.BlockSpec(memory_space=pltpu.VMEM),
            pl.BlockSpec(memory_space=pltpu.VMEM),
            pl.BlockSpec(memory_space=pltpu.MemorySpace.HBM),
            pl.BlockSpec(memory_space=pltpu.MemorySpace.HBM),
            pl.BlockSpec(memory_space=pltpu.VMEM),
        ],
        out_specs=pl.BlockSpec(memory_space=pltpu.VMEM),
        scratch_shapes=[
            pltpu.VMEM((Sq, HLDh), jnp.float32),
            pltpu.VMEM((2, Skv, 1, Dh), jnp.float32),
            pltpu.VMEM((2, Skv, 1, Dh), jnp.float32),
            pltpu.SemaphoreType.DMA((2,)),
            pltpu.SemaphoreType.DMA((2,)),
        ],
    )(x, Wq, K_ext, V_ext, Wo)

    CH = Sq // N_DEV
    HD = Dm // 2

    def ar_body(p_ref, out_ref,
                rsbufR, sendbufR, agbufR, redbufR,
                rsbufL, sendbufL, agbufL, redbufL,
                pbufR, pbufL,
                rs_ssR, rs_rsR, ag_ssR, ag_rsR,
                rs_ssL, rs_rsL, ag_ssL, ag_rsL):
        my = lax.axis_index("i")
        z = my // 4
        p = my % 4
        pos = jnp.where(p == 0, z,
              jnp.where(p == 3, 7 - z,
              jnp.where(p == 2, 8 + z, 15 - z)))

        def ring_logical(s):
            s = s % N_DEV
            seg = s // 4
            k = s % 4
            return jnp.where(seg == 0, 4 * k,
                   jnp.where(seg == 1, 4 * (3 - k) + 3,
                   jnp.where(seg == 2, 4 * k + 2, 4 * (3 - k) + 1)))

        left = ring_logical(pos + N_DEV - 1)
        right = ring_logical(pos + 1)

        barrier_sem = pltpu.get_barrier_semaphore()
        for nbr in (left, right):
            pl.semaphore_signal(
                barrier_sem, inc=1,
                device_id=(nbr,), device_id_type=pl.DeviceIdType.MESH)
        pl.semaphore_wait(barrier_sem, 2)

        bf16 = jnp.bfloat16
        f32 = jnp.float32
        pbufR[...] = p_ref[0, :, 0:HD].astype(bf16)
        pbufL[...] = p_ref[0, :, HD:Dm].astype(bf16)

        def rs_rdma(h):
            r = pltpu.make_async_remote_copy(
                src_ref=sendbufR.at[h], dst_ref=rsbufR.at[h],
                send_sem=rs_ssR.at[h], recv_sem=rs_rsR.at[h],
                device_id=(right,), device_id_type=pl.DeviceIdType.MESH)
            l = pltpu.make_async_remote_copy(
                src_ref=sendbufL.at[h], dst_ref=rsbufL.at[h],
                send_sem=rs_ssL.at[h], recv_sem=rs_rsL.at[h],
                device_id=(left,), device_id_type=pl.DeviceIdType.MESH)
            return r, l

        rs_desc = []
        sendbufR[0, :, :] = pbufR[pl.ds(pos * CH, CH), :]
        sendbufL[0, :, :] = pbufL[pl.ds(pos * CH, CH), :]
        d = rs_rdma(0)
        d[0].start()
        d[1].start()
        rs_desc.append(d)
        for h in range(N_DEV - 1):
            rs_desc[h][0].wait_recv()
            rs_desc[h][1].wait_recv()
            ckR = (pos + 2 * N_DEV - 1 - h) % N_DEV
            ckL = (pos + 1 + h) % N_DEV
            if h < N_DEV - 2:
                sendbufR[h + 1, :, :] = (
                    rsbufR[h] + pbufR[pl.ds(ckR * CH, CH), :])
                sendbufL[h + 1, :, :] = (
                    rsbufL[h] + pbufL[pl.ds(ckL * CH, CH), :])
                d = rs_rdma(h + 1)
                d[0].start()
                d[1].start()
                rs_desc.append(d)
            else:
                accR = (rsbufR[h].astype(f32)
                        + p_ref[0, pl.ds(ckR * CH, CH), 0:HD])
                accL = (rsbufL[h].astype(f32)
                        + p_ref[0, pl.ds(ckL * CH, CH), HD:Dm])
                redbufR[...] = accR.astype(bf16)
                redbufL[...] = accL.astype(bf16)

        def ag_rdma(h):
            r = pltpu.make_async_remote_copy(
                src_ref=(redbufR if h == 0 else agbufR.at[h - 1]),
                dst_ref=agbufR.at[h],
                send_sem=ag_ssR.at[h], recv_sem=ag_rsR.at[h],
                device_id=(right,), device_id_type=pl.DeviceIdType.MESH)
            l = pltpu.make_async_remote_copy(
                src_ref=(redbufL if h == 0 else agbufL.at[h - 1]),
                dst_ref=agbufL.at[h],
                send_sem=ag_ssL.at[h], recv_sem=ag_rsL.at[h],
                device_id=(left,), device_id_type=pl.DeviceIdType.MESH)
            return r, l

        ag_desc = []
        d = ag_rdma(0)
        d[0].start()
        d[1].start()
        ag_desc.append(d)
        ownR = (pos + 1) % N_DEV
        ownL = (pos + N_DEV - 1) % N_DEV
        out_ref[0, pl.ds(ownR * CH, CH), 0:HD] = accR
        out_ref[0, pl.ds(ownL * CH, CH), HD:Dm] = accL
        for h in range(N_DEV - 1):
            ag_desc[h][0].wait_recv()
            ag_desc[h][1].wait_recv()
            if h < N_DEV - 2:
                d = ag_rdma(h + 1)
                d[0].start()
                d[1].start()
                ag_desc.append(d)
            idxR = (pos + 2 * N_DEV - h) % N_DEV
            idxL = (pos + h) % N_DEV
            out_ref[0, pl.ds(idxR * CH, CH), 0:HD] = agbufR[h].astype(f32)
            out_ref[0, pl.ds(idxL * CH, CH), HD:Dm] = agbufL[h].astype(f32)

        for dR, dL in rs_desc + ag_desc:
            dR.wait_send()
            dL.wait_send()

    nh = N_DEV - 1
    return pl.pallas_call(
        ar_body,
        out_shape=jax.ShapeDtypeStruct((B, Sq, Dm), jnp.float32),
        in_specs=[pl.BlockSpec(memory_space=pltpu.VMEM)],
        out_specs=pl.BlockSpec(memory_space=pltpu.VMEM),
        scratch_shapes=[
            pltpu.VMEM((nh, CH, HD), jnp.bfloat16),
            pltpu.VMEM((nh, CH, HD), jnp.bfloat16),
            pltpu.VMEM((nh, CH, HD), jnp.bfloat16),
            pltpu.VMEM((CH, HD), jnp.bfloat16),
            pltpu.VMEM((nh, CH, HD), jnp.bfloat16),
            pltpu.VMEM((nh, CH, HD), jnp.bfloat16),
            pltpu.VMEM((nh, CH, HD), jnp.bfloat16),
            pltpu.VMEM((CH, HD), jnp.bfloat16),
            pltpu.VMEM((Sq, HD), jnp.bfloat16),
            pltpu.VMEM((Sq, HD), jnp.bfloat16),
            pltpu.SemaphoreType.DMA((nh,)),
            pltpu.SemaphoreType.DMA((nh,)),
            pltpu.SemaphoreType.DMA((nh,)),
            pltpu.SemaphoreType.DMA((nh,)),
            pltpu.SemaphoreType.DMA((nh,)),
            pltpu.SemaphoreType.DMA((nh,)),
            pltpu.SemaphoreType.DMA((nh,)),
            pltpu.SemaphoreType.DMA((nh,)),
        ],
        compiler_params=pltpu.CompilerParams(collective_id=0),
    )(partial)


# device time: 185903 ns/iter; 1.0645x vs baseline; 1.0608x over previous
import jax
import jax.numpy as jnp
from jax import lax
from jax.experimental import pallas as pl
from jax.experimental.pallas import tpu as pltpu

N_DEV = 16
SCALE = 0.08838834764831843
BLK = 64
QC = 512


def kernel(x, Wq, K_ext, V_ext, Wo):
    B, Sq, Dm = x.shape
    _, HLDh = Wq.shape
    _, Skv, Hq_g, Dh = K_ext.shape
    HL = HLDh // Dh
    NQC = Sq // QC

    def compute_body(x_ref, wq_ref, k_hbm, v_hbm, wo_ref, out_ref,
                     q_scr, k_scr, v_scr, ksems, vsems):
        my_i = lax.axis_index("i")

        def start_kv(hh):
            slot = hh % 2
            head = my_i * HL + hh
            ck = pltpu.make_async_copy(
                k_hbm.at[0, :, pl.ds(head, 1), :], k_scr.at[slot],
                ksems.at[slot])
            cv = pltpu.make_async_copy(
                v_hbm.at[0, :, pl.ds(head, 1), :], v_scr.at[slot],
                vsems.at[slot])
            ck.start()
            cv.start()
            return ck, cv

        x2d = x_ref[0]
        q_scr[...] = jnp.dot(x2d, wq_ref[...],
                             preferred_element_type=jnp.float32)
        out_ref[...] = jnp.zeros_like(out_ref)
        for h in range(HL):
            ck, cv = start_kv(h)
            ck.wait()
            cv.wait()
            slot = h % 2
            kh = k_scr[slot, :, 0, :]
            vh = v_scr[slot, :, 0, :]
            wo_blk = wo_ref[h * Dh:(h + 1) * Dh, :]
            for qc in range(NQC):
                kmax = (qc + 1) * QC
                qh = q_scr[qc * QC:(qc + 1) * QC, h * Dh:(h + 1) * Dh]
                s = lax.dot_general(
                    qh, kh[:kmax, :], (((1,), (1,)), ((), ())),
                    preferred_element_type=jnp.float32) * SCALE
                rows = (lax.broadcasted_iota(jnp.int32, (QC, kmax), 0)
                        + qc * QC) // BLK
                cols = lax.broadcasted_iota(jnp.int32, (QC, kmax), 1) // BLK
                s = jnp.where(cols <= rows, s, -1e9)
                m = jnp.max(s, axis=-1, keepdims=True)
                w = jnp.exp(s - m)
                w = w / jnp.sum(w, axis=-1, keepdims=True)
                ctx = jnp.dot(w, vh[:kmax, :],
                              preferred_element_type=jnp.float32)
                out_ref[0, qc * QC:(qc + 1) * QC, :] = (
                    out_ref[0, qc * QC:(qc + 1) * QC, :]
                    + jnp.dot(ctx, wo_blk,
                              preferred_element_type=jnp.float32))

    partial = pl.pallas_call(
        compute_body,
        out_shape=jax.ShapeDtypeStruct((B, Sq, Dm), jnp.float32),
        in_specs=[
            pl.BlockSpec(memory_space=pltpu.VMEM),
            pl.BlockSpec(memory_space=pltpu.VMEM),
            pl.BlockSpec(memory_space=pltpu.MemorySpace.HBM),
            pl.BlockSpec(memory_space=pltpu.MemorySpace.HBM),
            pl.BlockSpec(memory_space=pltpu.VMEM),
        ],
        out_specs=pl.BlockSpec(memory_space=pltpu.VMEM),
        scratch_shapes=[
            pltpu.VMEM((Sq, HLDh), jnp.float32),
            pltpu.VMEM((2, Skv, 1, Dh), jnp.float32),
            pltpu.VMEM((2, Skv, 1, Dh), jnp.float32),
            pltpu.SemaphoreType.DMA((2,)),
            pltpu.SemaphoreType.DMA((2,)),
        ],
    )(x, Wq, K_ext, V_ext, Wo)

    CH = Sq // N_DEV
    HD = Dm // 2

    def ar_body(p_ref, out_ref,
                rsbufR, sendbufR, agbufR, redbufR,
                rsbufL, sendbufL, agbufL, redbufL,
                pbufR, pbufL,
                rs_ssR, rs_rsR, ag_ssR, ag_rsR,
                rs_ssL, rs_rsL, ag_ssL, ag_rsL):
        my = lax.axis_index("i")
        z = my // 4
        p = my % 4
        pos = jnp.where(p == 0, z,
              jnp.where(p == 3, 7 - z,
              jnp.where(p == 2, 8 + z, 15 - z)))

        def ring_logical(s):
            s = s % N_DEV
            seg = s // 4
            k = s % 4
            return jnp.where(seg == 0, 4 * k,
                   jnp.where(seg == 1, 4 * (3 - k) + 3,
                   jnp.where(seg == 2, 4 * k + 2, 4 * (3 - k) + 1)))

        left = ring_logical(pos + N_DEV - 1)
        right = ring_logical(pos + 1)

        barrier_sem = pltpu.get_barrier_semaphore()
        for nbr in (left, right):
            pl.semaphore_signal(
                barrier_sem, inc=1,
                device_id=(nbr,), device_id_type=pl.DeviceIdType.MESH)
        pl.semaphore_wait(barrier_sem, 2)

        bf16 = jnp.bfloat16
        f32 = jnp.float32
        pbufR[...] = p_ref[0, :, 0:HD].astype(bf16)
        pbufL[...] = p_ref[0, :, HD:Dm].astype(bf16)

        def rs_rdma(h):
            r = pltpu.make_async_remote_copy(
                src_ref=sendbufR.at[h], dst_ref=rsbufR.at[h],
                send_sem=rs_ssR.at[h], recv_sem=rs_rsR.at[h],
                device_id=(right,), device_id_type=pl.DeviceIdType.MESH)
            l = pltpu.make_async_remote_copy(
                src_ref=sendbufL.at[h], dst_ref=rsbufL.at[h],
                send_sem=rs_ssL.at[h], recv_sem=rs_rsL.at[h],
                device_id=(left,), device_id_type=pl.DeviceIdType.MESH)
            return r, l

        rs_desc = []
        sendbufR[0, :, :] = pbufR[pl.ds(pos * CH, CH), :]
        sendbufL[0, :, :] = pbufL[pl.ds(pos * CH, CH), :]
        d = rs_rdma(0)
        d[0].start()
        d[1].start()
        rs_desc.append(d)
        for h in range(N_DEV - 1):
            rs_desc[h][0].wait_recv()
            rs_desc[h][1].wait_recv()
            ckR = (pos + 2 * N_DEV - 1 - h) % N_DEV
            ckL = (pos + 1 + h) % N_DEV
            if h < N_DEV - 2:
                sendbufR[h + 1, :, :] = (
                    rsbufR[h] + pbufR[pl.ds(ckR * CH, CH), :])
                sendbufL[h + 1, :, :] = (
                    rsbufL[h] + pbufL[pl.ds(ckL * CH, CH), :])
                d = rs_rdma(h + 1)
                d[0].start()
                d[1].start()
                rs_desc.append(d)
            else:
                accR = (rsbufR[h].astype(f32)
                        + p_ref[0, pl.ds(ckR * CH, CH), 0:HD])
                accL = (rsbufL[h].astype(f32)
                        + p_ref[0, pl.ds(ckL * CH, CH), HD:Dm])
                redbufR[...] = accR.astype(bf16)
                redbufL[...] = accL.astype(bf16)

        def ag_rdma(src, slot, bufs, ss, rs, dev):
            return pltpu.make_async_remote_copy(
                src_ref=src, dst_ref=bufs.at[slot],
                send_sem=ss.at[slot], recv_sem=rs.at[slot],
                device_id=(dev,), device_id_type=pl.DeviceIdType.MESH)

        ownR = (pos + 1) % N_DEV
        ownL = (pos + N_DEV - 1) % N_DEV
        ag_all = []
        for t in range(8):
            started = [
                ag_rdma(redbufR if t == 0 else agbufR.at[t - 1], t,
                        agbufR, ag_ssR, ag_rsR, right),
                ag_rdma(redbufL if t == 0 else agbufL.at[t - 1], t,
                        agbufL, ag_ssL, ag_rsL, left),
            ]
            if t < 7:
                started += [
                    ag_rdma(redbufR if t == 0 else agbufR.at[8 + t - 1],
                            8 + t, agbufR, ag_ssR, ag_rsR, left),
                    ag_rdma(redbufL if t == 0 else agbufL.at[8 + t - 1],
                            8 + t, agbufL, ag_ssL, ag_rsL, right),
                ]
            for d in started:
                d.start()
            ag_all += started
            if t == 0:
                out_ref[0, pl.ds(ownR * CH, CH), 0:HD] = accR
                out_ref[0, pl.ds(ownL * CH, CH), HD:Dm] = accL
            for d in started:
                d.wait_recv()
            idxRr = (pos + 2 * N_DEV - t) % N_DEV
            idxLl = (pos + t) % N_DEV
            out_ref[0, pl.ds(idxRr * CH, CH), 0:HD] = agbufR[t].astype(f32)
            out_ref[0, pl.ds(idxLl * CH, CH), HD:Dm] = agbufL[t].astype(f32)
            if t < 7:
                idxRl = (pos + t + 2) % N_DEV
                idxLr = (pos + 2 * N_DEV - t - 2) % N_DEV
                out_ref[0, pl.ds(idxRl * CH, CH), 0:HD] = (
                    agbufR[8 + t].astype(f32))
                out_ref[0, pl.ds(idxLr * CH, CH), HD:Dm] = (
                    agbufL[8 + t].astype(f32))

        for dR, dL in rs_desc:
            dR.wait_send()
            dL.wait_send()
        for d in ag_all:
            d.wait_send()

    nh = N_DEV - 1
    return pl.pallas_call(
        ar_body,
        out_shape=jax.ShapeDtypeStruct((B, Sq, Dm), jnp.float32),
        in_specs=[pl.BlockSpec(memory_space=pltpu.VMEM)],
        out_specs=pl.BlockSpec(memory_space=pltpu.VMEM),
        scratch_shapes=[
            pltpu.VMEM((nh, CH, HD), jnp.bfloat16),
            pltpu.VMEM((nh, CH, HD), jnp.bfloat16),
            pltpu.VMEM((nh, CH, HD), jnp.bfloat16),
            pltpu.VMEM((CH, HD), jnp.bfloat16),
            pltpu.VMEM((nh, CH, HD), jnp.bfloat16),
            pltpu.VMEM((nh, CH, HD), jnp.bfloat16),
            pltpu.VMEM((nh, CH, HD), jnp.bfloat16),
            pltpu.VMEM((CH, HD), jnp.bfloat16),
            pltpu.VMEM((Sq, HD), jnp.bfloat16),
            pltpu.VMEM((Sq, HD), jnp.bfloat16),
            pltpu.SemaphoreType.DMA((nh,)),
            pltpu.SemaphoreType.DMA((nh,)),
            pltpu.SemaphoreType.DMA((nh,)),
            pltpu.SemaphoreType.DMA((nh,)),
            pltpu.SemaphoreType.DMA((nh,)),
            pltpu.SemaphoreType.DMA((nh,)),
            pltpu.SemaphoreType.DMA((nh,)),
            pltpu.SemaphoreType.DMA((nh,)),
        ],
        compiler_params=pltpu.CompilerParams(collective_id=0),
    )(partial)
